# Initial kernel scaffold; baseline (speedup 1.0000x reference)
import jax
import jax.numpy as jnp
from jax.experimental import pallas as pl


def kernel(x, edge_index, embedding, W, bias, att_i, att_j, att_em_i, att_em_j, bn_gamma, bn_beta):
    raise NotImplementedError("write your pallas kernel here")


# speedup vs baseline: 17.0250x; 17.0250x over previous
import functools

import jax
import jax.numpy as jnp
from jax import lax
from jax.experimental import pallas as pl
from jax.experimental.pallas import tpu as pltpu
from jax.experimental.pallas import tpu_sc as plsc

_N = 10000
_C = 128
_E = 320000
_NC = 2
_NS = 16
_NW = _NC * _NS
_CHUNK = 128
_NCHUNK = -(-(_E // _NW) // _CHUNK)
_EPW = _NCHUNK * _CHUNK
_EPAD = _EPW * _NW
_NP = 10240
_RPT = _NP // _NS
_RCP = 128
_NCOPY = _RPT // _RCP


def _prep_body(x_ref, emb_ref, Wt_ref, ati_ref, atei_ref, atj_ref, atej_ref,
               xl_ref, ai_ref, aj_ref, ajm_ref, ws_ref):
    dn = (((1,), (0,)), ((), ()))
    xl = lax.dot_general(x_ref[...], Wt_ref[...], dn,
                         preferred_element_type=jnp.float32)
    xl_ref[...] = xl
    emb = emb_ref[...]
    ai = (lax.dot_general(xl, ati_ref[...], dn, preferred_element_type=jnp.float32)
          + lax.dot_general(emb, atei_ref[...], dn, preferred_element_type=jnp.float32))
    aj = (lax.dot_general(xl, atj_ref[...], dn, preferred_element_type=jnp.float32)
          + lax.dot_general(emb, atej_ref[...], dn, preferred_element_type=jnp.float32))
    ajmax = jnp.max(aj)
    cb = ai + ajmax
    cb = jnp.maximum(cb, 0.2 * cb)
    sl = ai + aj
    sl = jnp.maximum(sl, 0.2 * sl)
    ai_ref[...] = ai
    aj_ref[...] = aj
    ajm_ref[...] = jnp.full((8, 16), ajmax, jnp.float32)
    ws_ref[...] = jnp.exp(sl - cb)


def _prep(x, emb, Wt, ati, atei, atj, atej):
    return pl.pallas_call(
        _prep_body,
        out_shape=[
            jax.ShapeDtypeStruct((_N, _C), jnp.float32),
            jax.ShapeDtypeStruct((_N, 1), jnp.float32),
            jax.ShapeDtypeStruct((_N, 1), jnp.float32),
            jax.ShapeDtypeStruct((8, 16), jnp.float32),
            jax.ShapeDtypeStruct((_N, 1), jnp.float32),
        ],
    )(x, emb, Wt, ati, atei, atj, atej)


def _sc_edge(src, dst, ai, aj, ajm, xl):
    mesh = plsc.VectorSubcoreMesh(core_axis_name="c", subcore_axis_name="s")

    @functools.partial(
        pl.kernel,
        mesh=mesh,
        compiler_params=pltpu.CompilerParams(needs_layout_passes=False),
        out_type=[
            jax.ShapeDtypeStruct((_NC * _NP, _C), jnp.float32),
            jax.ShapeDtypeStruct((_NW * _N,), jnp.float32),
        ],
        scratch_types=[
            pltpu.VMEM((_N,), jnp.float32),
            pltpu.VMEM((_N,), jnp.float32),
            pltpu.VMEM((16,), jnp.float32),
            pltpu.VMEM((_N,), jnp.float32),
            pltpu.VMEM((_CHUNK,), jnp.int32),
            pltpu.VMEM((_CHUNK,), jnp.int32),
            pltpu.VMEM((_CHUNK,), jnp.float32),
            pltpu.VMEM((_CHUNK, _C), jnp.float32),
            pltpu.VMEM_SHARED((_NP, _C), jnp.float32),
            pltpu.SemaphoreType.DMA,
        ],
    )
    def body(src_h, dst_h, ai_h, aj_h, ajm_h, xl_h, out_h, den_h,
             ai_v, aj_v, ajm_v, den_v, sidx, didx, w_v, rows, acc, sem):
        cid = lax.axis_index("c")
        sid = lax.axis_index("s")
        wid = sid * _NC + cid
        pltpu.sync_copy(ai_h, ai_v)
        pltpu.sync_copy(aj_h, aj_v)
        pltpu.sync_copy(ajm_h, ajm_v)

        zero16 = jnp.zeros((16,), jnp.float32)

        def zden(i, carry):
            den_v[pl.ds(i * 16, 16)] = zero16
            return carry
        lax.fori_loop(0, _N // 16, zden, 0)

        def zrow(e, carry):
            for k in range(_C // 16):
                rows[e, pl.ds(k * 16, 16)] = zero16
            return carry
        lax.fori_loop(0, _CHUNK, zrow, 0)

        rbase = sid * _RPT
        for j in range(_NCOPY):
            pltpu.sync_copy(rows.at[pl.ds(0, _RCP)],
                            acc.at[pl.ds(rbase + j * _RCP, _RCP)])
        plsc.subcore_barrier()

        ebase = wid * _EPW

        def chunk(ic, carry):
            base = ebase + ic * _CHUNK
            pltpu.sync_copy(src_h.at[pl.ds(base, _CHUNK)], sidx)
            pltpu.sync_copy(dst_h.at[pl.ds(base, _CHUNK)], didx)
            ajm16 = ajm_v[...]
            for j in range(_CHUNK // 16):
                s16 = sidx[pl.ds(j * 16, 16)]
                d16 = didx[pl.ds(j * 16, 16)]
                vs = plsc.load_gather(aj_v, [s16])
                ud = plsc.load_gather(ai_v, [d16])
                am = ud + ajm16
                cd = jnp.maximum(am, 0.2 * am)
                al = ud + vs
                lr = jnp.maximum(al, 0.2 * al)
                w = jnp.where(s16 != d16, jnp.exp(lr - cd), 0.0)
                w_v[pl.ds(j * 16, 16)] = w
                plsc.addupdate_scatter(den_v, [d16], w)
            pltpu.async_copy(xl_h.at[sidx], rows, sem).wait()

            def scale(e, c2):
                we = plsc.load_gather(w_v, [jnp.full((16,), e, jnp.int32)])
                for k in range(_C // 16):
                    rows[e, pl.ds(k * 16, 16)] = rows[e, pl.ds(k * 16, 16)] * we
                return c2
            lax.fori_loop(0, _CHUNK, scale, 0)
            pltpu.sync_copy(rows, acc.at[didx], add=True)
            return carry
        lax.fori_loop(0, _NCHUNK, chunk, 0)

        plsc.subcore_barrier()
        for j in range(_NCOPY):
            pltpu.sync_copy(acc.at[pl.ds(rbase + j * _RCP, _RCP)],
                            out_h.at[pl.ds(cid * _NP + rbase + j * _RCP, _RCP)])
        pltpu.sync_copy(den_v, den_h.at[pl.ds(wid * _N, _N)])

    return body(src, dst, ai, aj, ajm, xl)


def _fin_body(p_ref, dpT_ref, ws_ref, xl_ref, b_ref, g_ref, be_ref, o_ref):
    num = p_ref[0] + p_ref[1] + ws_ref[...] * xl_ref[...]
    ones = jnp.ones((_NW, 1), jnp.float32)
    den = lax.dot_general(dpT_ref[...], ones, (((1,), (0,)), ((), ())),
                          preferred_element_type=jnp.float32)
    den = den + ws_ref[...] + 1e-16
    out = num / den + b_ref[...]
    mu = jnp.mean(out, axis=0, keepdims=True)
    var = jnp.mean(out * out, axis=0, keepdims=True) - mu * mu
    inv = lax.rsqrt(var + 1e-5)
    o_ref[...] = jnp.maximum((out - mu) * inv * g_ref[...] + be_ref[...], 0.0)


def _fin(parts, dpT, ws, xl, b, g, be):
    return pl.pallas_call(
        _fin_body,
        out_shape=jax.ShapeDtypeStruct((_N, _C), jnp.float32),
    )(parts, dpT, ws, xl, b, g, be)


def kernel(x, edge_index, embedding, W, bias, att_i, att_j, att_em_i,
           att_em_j, bn_gamma, bn_beta):
    ati = att_i.reshape(_C, 1)
    atei = att_em_i.reshape(_C, 1)
    atj = att_j.reshape(_C, 1)
    atej = att_em_j.reshape(_C, 1)
    xl, ai, aj, ajm, ws = _prep(x, embedding, W.T, ati, atei, atj, atej)
    pad = _EPAD - _E
    src = jnp.concatenate([edge_index[0], jnp.zeros((pad,), jnp.int32)])
    dst = jnp.concatenate([edge_index[1], jnp.zeros((pad,), jnp.int32)])
    parts, den1 = _sc_edge(src, dst, ai.reshape(_N), aj.reshape(_N),
                           ajm.reshape(128)[:16], xl)
    parts = parts.reshape(_NC, _NP, _C)[:, :_N, :]
    dpT = den1.reshape(_NW, _N).T
    out = _fin(parts, dpT, ws, xl, bias.reshape(1, _C),
               bn_gamma.reshape(1, _C), bn_beta.reshape(1, _C))
    return out
